# baseline (speedup 1.0000x reference)
import functools

import jax
import jax.numpy as jnp
from jax import lax
from jax.experimental import pallas as pl
from jax.experimental.pallas import tpu as pltpu
from jax.experimental.pallas import tpu_sc as plsc

B = 16384
D = 64
CHUNK = 128
NC, NS = 2, 16
NW = NC * NS
BPW = B // NW
CH = BPW // CHUNK

_mesh = plsc.VectorSubcoreMesh(core_axis_name="c", subcore_axis_name="s")


@functools.partial(
    pl.kernel,
    out_type=jax.ShapeDtypeStruct((B,), jnp.float32),
    mesh=_mesh,
    compiler_params=pltpu.CompilerParams(use_tc_tiling_on_sc=False),
    scratch_types=[
        pltpu.VMEM((CH, CHUNK), jnp.int32),
        pltpu.VMEM((CH, CHUNK), jnp.int32),
        pltpu.VMEM((CH, CHUNK), jnp.int32),
        pltpu.VMEM((BPW, D), jnp.float32),
        pltpu.VMEM((BPW, D), jnp.float32),
        pltpu.VMEM((BPW, D), jnp.float32),
        pltpu.VMEM((BPW,), jnp.float32),
        pltpu.SemaphoreType.DMA,
    ],
)
def _simple_score(h_idx, t_idx, r_idx, entity, relation, out,
                  hidx_v, tidx_v, ridx_v, hrow, trow, rrow, outv, sem):
    wid = lax.axis_index("s") * NC + lax.axis_index("c")
    row0 = wid * CH

    pltpu.sync_copy(h_idx.at[pl.ds(row0, CH)], hidx_v)
    pltpu.sync_copy(t_idx.at[pl.ds(row0, CH)], tidx_v)
    pltpu.sync_copy(r_idx.at[pl.ds(row0, CH)], ridx_v)

    copies = []
    for j in range(CH):
        dst = pl.ds(j * CHUNK, CHUNK)
        copies.append(pltpu.async_copy(entity.at[hidx_v.at[j]], hrow.at[dst], sem))
        copies.append(pltpu.async_copy(entity.at[tidx_v.at[j]], trow.at[dst], sem))
        copies.append(pltpu.async_copy(relation.at[ridx_v.at[j]], rrow.at[dst], sem))
    for c in copies:
        c.wait()

    lane = lax.iota(jnp.int32, 16)
    dnums = lax.GatherDimensionNumbers(
        offset_dims=(), collapsed_slice_dims=(0,), start_index_map=(0,))
    perms = [(lane ^ s)[:, None] for s in (8, 4, 2, 1)]

    def lanesum(v):
        for p in perms:
            v = v + lax.gather(v, p, dnums, (1,),
                               mode=lax.GatherScatterMode.PROMISE_IN_BOUNDS)
        return v

    nsl = D // 16

    def group(g, _):
        sv = jnp.zeros((16,), jnp.float32)
        for i in range(16):
            b = g * 16 + i
            h = [hrow[b, pl.ds(16 * k, 16)] for k in range(nsl)]
            r = [rrow[b, pl.ds(16 * k, 16)] for k in range(nsl)]
            t = [trow[b, pl.ds(16 * k, 16)] for k in range(nsl)]
            acc = h[0] * r[0] * t[2]
            acc = acc + h[1] * r[1] * t[3]
            acc = acc + h[2] * r[2] * t[0]
            acc = acc + h[3] * r[3] * t[1]
            sv = jnp.where(lane == i, lanesum(acc), sv)
        outv[pl.ds(g * 16, 16)] = sv
        return 0

    lax.fori_loop(0, BPW // 16, group, 0)

    pltpu.sync_copy(outv, out.at[pl.ds(wid * BPW, BPW)])


def kernel(graph, h_index, t_index, r_index, entity, relation):
    hi = h_index.astype(jnp.int32).reshape(B // CHUNK, CHUNK)
    ti = t_index.astype(jnp.int32).reshape(B // CHUNK, CHUNK)
    ri = r_index.astype(jnp.int32).reshape(B // CHUNK, CHUNK)
    return _simple_score(hi, ti, ri, entity, relation)


# speedup vs baseline: 1.1024x; 1.1024x over previous
import functools

import jax
import jax.numpy as jnp
from jax import lax
from jax.experimental import pallas as pl
from jax.experimental.pallas import tpu as pltpu
from jax.experimental.pallas import tpu_sc as plsc

B = 16384
D = 64
PD = 2 * D
CHUNK = 128
NC, NS = 2, 16
NW = NC * NS
BPW = B // NW
CH = BPW // CHUNK

_mesh = plsc.VectorSubcoreMesh(core_axis_name="c", subcore_axis_name="s")


@functools.partial(
    pl.kernel,
    out_type=jax.ShapeDtypeStruct((B,), jnp.float32),
    mesh=_mesh,
    compiler_params=pltpu.CompilerParams(use_tc_tiling_on_sc=False),
    scratch_types=[
        pltpu.VMEM((CH, CHUNK), jnp.int32),
        pltpu.VMEM((CH, CHUNK), jnp.int32),
        pltpu.VMEM((CH, CHUNK), jnp.int32),
        pltpu.VMEM((2, CHUNK, PD), jnp.float32),
        pltpu.VMEM((2, CHUNK, PD), jnp.float32),
        pltpu.VMEM((2, CHUNK, PD), jnp.float32),
        pltpu.VMEM((BPW,), jnp.float32),
        pltpu.SemaphoreType.DMA,
        pltpu.SemaphoreType.DMA,
    ],
)
def _simple_score(h_idx, t_idx, r_idx, entp, relp, out,
                  hidx_v, tidx_v, ridx_v, hbuf, tbuf, rbuf, outv, sem0, sem1):
    wid = lax.axis_index("s") * NC + lax.axis_index("c")
    row0 = wid * CH

    pltpu.sync_copy(h_idx.at[pl.ds(row0, CH)], hidx_v)
    pltpu.sync_copy(t_idx.at[pl.ds(row0, CH)], tidx_v)
    pltpu.sync_copy(r_idx.at[pl.ds(row0, CH)], ridx_v)

    sems = (sem0, sem1)

    def fire(j):
        p = j % 2
        return [
            pltpu.async_copy(entp.at[hidx_v.at[j]], hbuf.at[p], sems[p]),
            pltpu.async_copy(entp.at[tidx_v.at[j]], tbuf.at[p], sems[p]),
            pltpu.async_copy(relp.at[ridx_v.at[j]], rbuf.at[p], sems[p]),
        ]

    lane = lax.iota(jnp.int32, 16)
    dnums = lax.GatherDimensionNumbers(
        offset_dims=(), collapsed_slice_dims=(0,), start_index_map=(0,))
    perms = [(lane ^ s)[:, None] for s in (8, 4, 2, 1)]

    def lanesum(v):
        for p in perms:
            v = v + lax.gather(v, p, dnums, (1,),
                               mode=lax.GatherScatterMode.PROMISE_IN_BOUNDS)
        return v

    nsl = D // 16

    def compute(j):
        p = j % 2

        def group(g, _):
            sv = jnp.zeros((16,), jnp.float32)
            for i in range(16):
                b = g * 16 + i
                h = [hbuf[p, b, pl.ds(16 * k, 16)] for k in range(nsl)]
                r = [rbuf[p, b, pl.ds(16 * k, 16)] for k in range(nsl)]
                t = [tbuf[p, b, pl.ds(16 * k, 16)] for k in range(nsl)]
                acc = h[0] * r[0] * t[2]
                acc = acc + h[1] * r[1] * t[3]
                acc = acc + h[2] * r[2] * t[0]
                acc = acc + h[3] * r[3] * t[1]
                sv = jnp.where(lane == i, lanesum(acc), sv)
            outv[pl.ds(j * CHUNK + g * 16, 16)] = sv
            return 0

        lax.fori_loop(0, CHUNK // 16, group, 0)

    inflight = {0: fire(0), 1: fire(1)}
    for j in range(CH):
        for c in inflight.pop(j):
            c.wait()
        compute(j)
        if j + 2 < CH:
            inflight[j + 2] = fire(j + 2)

    pltpu.sync_copy(outv, out.at[pl.ds(wid * BPW, BPW)])


def kernel(graph, h_index, t_index, r_index, entity, relation):
    entp = jnp.pad(entity, ((0, 0), (0, PD - D)))
    relp = jnp.pad(relation, ((0, 0), (0, PD - D)))
    hi = h_index.astype(jnp.int32).reshape(B // CHUNK, CHUNK)
    ti = t_index.astype(jnp.int32).reshape(B // CHUNK, CHUNK)
    ri = r_index.astype(jnp.int32).reshape(B // CHUNK, CHUNK)
    return _simple_score(hi, ti, ri, entp, relp)


# speedup vs baseline: 1.6737x; 1.5182x over previous
import functools

import jax
import jax.numpy as jnp
from jax import lax
from jax.experimental import pallas as pl
from jax.experimental.pallas import tpu as pltpu
from jax.experimental.pallas import tpu_sc as plsc

B = 16384
D = 64
CHUNK = 128
NC, NS = 2, 16
NW = NC * NS
BPW = B // NW
CH = BPW // CHUNK

_mesh = plsc.VectorSubcoreMesh(core_axis_name="c", subcore_axis_name="s")


@functools.partial(
    pl.kernel,
    out_type=jax.ShapeDtypeStruct((B,), jnp.float32),
    mesh=_mesh,
    compiler_params=pltpu.CompilerParams(use_tc_tiling_on_sc=True),
    scratch_types=[
        pltpu.VMEM((BPW,), jnp.int32),
        pltpu.VMEM((BPW,), jnp.int32),
        pltpu.VMEM((BPW,), jnp.int32),
        pltpu.VMEM((2, CHUNK, D), jnp.float32),
        pltpu.VMEM((2, CHUNK, D), jnp.float32),
        pltpu.VMEM((2, CHUNK, D), jnp.float32),
        pltpu.VMEM((BPW,), jnp.float32),
        pltpu.SemaphoreType.DMA,
        pltpu.SemaphoreType.DMA,
    ],
)
def _simple_score(h_idx, t_idx, r_idx, entity, relation, out,
                  hidx_v, tidx_v, ridx_v, hbuf, tbuf, rbuf, outv, sem0, sem1):
    wid = lax.axis_index("s") * NC + lax.axis_index("c")
    base = wid * BPW

    pltpu.sync_copy(h_idx.at[pl.ds(base, BPW)], hidx_v)
    pltpu.sync_copy(t_idx.at[pl.ds(base, BPW)], tidx_v)
    pltpu.sync_copy(r_idx.at[pl.ds(base, BPW)], ridx_v)

    sems = (sem0, sem1)

    def fire(j):
        p = j % 2
        sem = sems[p]

        def grp(g, _):
            hv = hidx_v[pl.ds(j * CHUNK + g * 16, 16)]
            tv = tidx_v[pl.ds(j * CHUNK + g * 16, 16)]
            rv = ridx_v[pl.ds(j * CHUNK + g * 16, 16)]
            for i in range(16):
                b = g * 16 + i
                pltpu.async_copy(entity.at[hv[i]], hbuf.at[p, b], sem)
                pltpu.async_copy(entity.at[tv[i]], tbuf.at[p, b], sem)
                pltpu.async_copy(relation.at[rv[i]], rbuf.at[p, b], sem)
            return 0

        lax.fori_loop(0, CHUNK // 16, grp, 0)

    def drain(j):
        p = j % 2
        sem = sems[p]
        pltpu.make_async_copy(entity.at[pl.ds(0, CHUNK)], hbuf.at[p], sem).wait()
        pltpu.make_async_copy(entity.at[pl.ds(0, CHUNK)], tbuf.at[p], sem).wait()
        pltpu.make_async_copy(entity.at[pl.ds(0, CHUNK)], rbuf.at[p], sem).wait()

    lane = lax.iota(jnp.int32, 16)
    dnums = lax.GatherDimensionNumbers(
        offset_dims=(), collapsed_slice_dims=(0,), start_index_map=(0,))
    perms = [(lane ^ s)[:, None] for s in (8, 4, 2, 1)]

    def lanesum(v):
        for p in perms:
            v = v + lax.gather(v, p, dnums, (1,),
                               mode=lax.GatherScatterMode.PROMISE_IN_BOUNDS)
        return v

    nsl = D // 16

    def compute(j):
        p = j % 2

        def group(g, _):
            sv = jnp.zeros((16,), jnp.float32)
            for i in range(16):
                b = g * 16 + i
                h = [hbuf[p, b, pl.ds(16 * k, 16)] for k in range(nsl)]
                r = [rbuf[p, b, pl.ds(16 * k, 16)] for k in range(nsl)]
                t = [tbuf[p, b, pl.ds(16 * k, 16)] for k in range(nsl)]
                acc = h[0] * r[0] * t[2]
                acc = acc + h[1] * r[1] * t[3]
                acc = acc + h[2] * r[2] * t[0]
                acc = acc + h[3] * r[3] * t[1]
                sv = jnp.where(lane == i, lanesum(acc), sv)
            outv[pl.ds(j * CHUNK + g * 16, 16)] = sv
            return 0

        lax.fori_loop(0, CHUNK // 16, group, 0)

    fire(0)
    fire(1)
    for j in range(CH):
        drain(j)
        compute(j)
        if j + 2 < CH:
            fire(j + 2)

    pltpu.sync_copy(outv, out.at[pl.ds(base, BPW)])


def kernel(graph, h_index, t_index, r_index, entity, relation):
    hi = h_index.astype(jnp.int32)
    ti = t_index.astype(jnp.int32)
    ri = r_index.astype(jnp.int32)
    return _simple_score(hi, ti, ri, entity, relation)


# speedup vs baseline: 1.6750x; 1.0008x over previous
import functools

import jax
import jax.numpy as jnp
from jax import lax
from jax.experimental import pallas as pl
from jax.experimental.pallas import tpu as pltpu
from jax.experimental.pallas import tpu_sc as plsc

B = 16384
D = 64
E = 1000000
CHUNK = 128
NC, NS = 2, 16
NW = NC * NS
BPW = B // NW
CH = BPW // CHUNK

_mesh = plsc.VectorSubcoreMesh(core_axis_name="c", subcore_axis_name="s")

_TBN = 4096


def _transpose_block(x_ref, o_ref):
    o_ref[...] = jnp.swapaxes(x_ref[...], 0, 1)


_format_table = pl.pallas_call(
    _transpose_block,
    grid=(pl.cdiv(E, _TBN),),
    in_specs=[pl.BlockSpec((D, _TBN), lambda i: (0, i))],
    out_specs=pl.BlockSpec((_TBN, D), lambda i: (i, 0)),
    out_shape=jax.ShapeDtypeStruct((E, D), jnp.float32),
)


@functools.partial(
    pl.kernel,
    out_type=jax.ShapeDtypeStruct((B,), jnp.float32),
    mesh=_mesh,
    compiler_params=pltpu.CompilerParams(use_tc_tiling_on_sc=True),
    scratch_types=[
        pltpu.VMEM((BPW,), jnp.int32),
        pltpu.VMEM((BPW,), jnp.int32),
        pltpu.VMEM((BPW,), jnp.int32),
        pltpu.VMEM((2, CHUNK, D), jnp.float32),
        pltpu.VMEM((2, CHUNK, D), jnp.float32),
        pltpu.VMEM((2, CHUNK, D), jnp.float32),
        pltpu.VMEM((BPW,), jnp.float32),
        pltpu.SemaphoreType.DMA,
        pltpu.SemaphoreType.DMA,
    ],
)
def _simple_score(h_idx, t_idx, r_idx, entity, relation, out,
                  hidx_v, tidx_v, ridx_v, hbuf, tbuf, rbuf, outv, sem0, sem1):
    wid = lax.axis_index("s") * NC + lax.axis_index("c")
    base = wid * BPW

    pltpu.sync_copy(h_idx.at[pl.ds(base, BPW)], hidx_v)
    pltpu.sync_copy(t_idx.at[pl.ds(base, BPW)], tidx_v)
    pltpu.sync_copy(r_idx.at[pl.ds(base, BPW)], ridx_v)

    sems = (sem0, sem1)

    def fire(j):
        p = j % 2
        sem = sems[p]

        def grp(g, _):
            hv = hidx_v[pl.ds(j * CHUNK + g * 16, 16)]
            tv = tidx_v[pl.ds(j * CHUNK + g * 16, 16)]
            rv = ridx_v[pl.ds(j * CHUNK + g * 16, 16)]
            for i in range(16):
                b = g * 16 + i
                pltpu.async_copy(entity.at[hv[i]], hbuf.at[p, b], sem)
                pltpu.async_copy(entity.at[tv[i]], tbuf.at[p, b], sem)
                pltpu.async_copy(relation.at[rv[i]], rbuf.at[p, b], sem)
            return 0

        lax.fori_loop(0, CHUNK // 16, grp, 0)

    def drain(j):
        p = j % 2
        sem = sems[p]
        pltpu.make_async_copy(entity.at[pl.ds(0, CHUNK)], hbuf.at[p], sem).wait()
        pltpu.make_async_copy(entity.at[pl.ds(0, CHUNK)], tbuf.at[p], sem).wait()
        pltpu.make_async_copy(entity.at[pl.ds(0, CHUNK)], rbuf.at[p], sem).wait()

    lane = lax.iota(jnp.int32, 16)
    dnums = lax.GatherDimensionNumbers(
        offset_dims=(), collapsed_slice_dims=(0,), start_index_map=(0,))
    perms = [(lane ^ s)[:, None] for s in (8, 4, 2, 1)]

    def lanesum(v):
        for p in perms:
            v = v + lax.gather(v, p, dnums, (1,),
                               mode=lax.GatherScatterMode.PROMISE_IN_BOUNDS)
        return v

    nsl = D // 16

    def compute(j):
        p = j % 2

        def group(g, _):
            sv = jnp.zeros((16,), jnp.float32)
            for i in range(16):
                b = g * 16 + i
                h = [hbuf[p, b, pl.ds(16 * k, 16)] for k in range(nsl)]
                r = [rbuf[p, b, pl.ds(16 * k, 16)] for k in range(nsl)]
                t = [tbuf[p, b, pl.ds(16 * k, 16)] for k in range(nsl)]
                acc = h[0] * r[0] * t[2]
                acc = acc + h[1] * r[1] * t[3]
                acc = acc + h[2] * r[2] * t[0]
                acc = acc + h[3] * r[3] * t[1]
                sv = jnp.where(lane == i, lanesum(acc), sv)
            outv[pl.ds(j * CHUNK + g * 16, 16)] = sv
            return 0

        lax.fori_loop(0, CHUNK // 16, group, 0)

    fire(0)
    fire(1)
    for j in range(CH):
        drain(j)
        compute(j)
        if j + 2 < CH:
            fire(j + 2)

    pltpu.sync_copy(outv, out.at[pl.ds(base, BPW)])


def kernel(graph, h_index, t_index, r_index, entity, relation):
    hi = h_index.astype(jnp.int32)
    ti = t_index.astype(jnp.int32)
    ri = r_index.astype(jnp.int32)
    entity_fmt = _format_table(jnp.swapaxes(entity, 0, 1))
    return _simple_score(hi, ti, ri, entity_fmt, relation)
